# Initial kernel scaffold; baseline (speedup 1.0000x reference)
import jax
import jax.numpy as jnp
from jax.experimental import pallas as pl


def kernel(x, anchors):
    raise NotImplementedError("write your pallas kernel here")


# speedup vs baseline: 1.2213x; 1.2213x over previous
import functools

import jax
import jax.numpy as jnp
from jax import lax
from jax.experimental import pallas as pl
from jax.experimental.pallas import tpu as pltpu
from jax.experimental.pallas import tpu_sc as plsc

K_ANCH = 100000
D = 768
BQ = 64
T = 512
TOPK = 16
BLK = 2048
NB = (K_ANCH + BLK - 1) // BLK
IMIN = jnp.iinfo(jnp.int32).min
NEG = -3.0e38


def _mono(f):
    b = lax.bitcast_convert_type(f, jnp.int32)
    return jnp.where(b < 0, b ^ 0x7FFFFFFF, b)


def _unmono(b):
    return lax.bitcast_convert_type(
        jnp.where(b < 0, b ^ 0x7FFFFFFF, b), jnp.float32
    )


def _qsum_body(x_ref, o_ref):
    o_ref[...] = jnp.sum(x_ref[...], axis=1)


def _topk_body(qs_ref, a_ref, o_ref, rv_ref, ri_ref):
    i = pl.program_id(0)

    @pl.when(i == 0)
    def _init():
        rv_ref[...] = jnp.full((BQ, TOPK), NEG, jnp.float32)
        ri_ref[...] = jnp.zeros((BQ, TOPK), jnp.int32)

    a = a_ref[...]
    ss = jnp.sum(a * a, axis=1, keepdims=True)
    inv = 1.0 / jnp.maximum(jnp.sqrt(ss), 1e-12)
    ab = (a * inv).astype(jnp.bfloat16)
    qb = qs_ref[...].astype(jnp.bfloat16)
    sims = lax.dot_general(
        qb, ab, (((1,), (1,)), ((), ())), preferred_element_type=jnp.float32
    )
    lane = lax.broadcasted_iota(jnp.int32, (BQ, BLK), 1)
    sims = jnp.where(i * BLK + lane < K_ANCH, sims, NEG)

    keys = (_mono(sims) & ~(BLK - 1)) | (BLK - 1 - lane)
    bks = []
    for _ in range(TOPK):
        m = jnp.max(keys, axis=1, keepdims=True)
        keys = jnp.where(keys == m, IMIN, keys)
        bks.append(m)
    bk = jnp.concatenate(bks, axis=1)
    bi = i * BLK + (BLK - 1 - (bk & (BLK - 1)))
    bv = _unmono(bk & ~(BLK - 1))

    cv = jnp.concatenate([rv_ref[...], bv], axis=1)
    ci = jnp.concatenate([ri_ref[...], bi], axis=1)
    pos = lax.broadcasted_iota(jnp.int32, (BQ, 2 * TOPK), 1)
    mk = (_mono(cv) & ~31) | (31 - pos)
    nvs, nis = [], []
    for _ in range(TOPK):
        m = jnp.max(mk, axis=1, keepdims=True)
        hit = mk == m
        mk = jnp.where(hit, IMIN, mk)
        nvs.append(jnp.sum(jnp.where(hit, cv, 0.0), axis=1, keepdims=True))
        nis.append(jnp.sum(jnp.where(hit, ci, 0), axis=1, keepdims=True))
    rv_ref[...] = jnp.concatenate(nvs, axis=1)
    ri_ref[...] = jnp.concatenate(nis, axis=1)

    @pl.when(i == NB - 1)
    def _emit():
        o_ref[...] = jnp.concatenate(
            [ri_ref[...], jnp.zeros((BQ, 128 - TOPK), jnp.int32)], axis=1
        )


def _sc_gather(table, idx_flat):
    n = idx_flat.shape[0]
    bpw = n // 32
    mesh = plsc.VectorSubcoreMesh(core_axis_name="c", subcore_axis_name="s")

    @functools.partial(
        pl.kernel,
        mesh=mesh,
        out_type=jax.ShapeDtypeStruct((n, D), jnp.float32),
        scratch_types=[
            pltpu.VMEM((bpw,), jnp.int32),
            pltpu.VMEM((bpw, D), jnp.float32),
            pltpu.SemaphoreType.DMA,
        ],
    )
    def k(table_hbm, idx_hbm, out_hbm, idx_v, rows_v, sem):
        wid = lax.axis_index("s") * 2 + lax.axis_index("c")
        base = wid * bpw
        pltpu.sync_copy(idx_hbm.at[pl.ds(base, bpw)], idx_v)
        pltpu.async_copy(table_hbm.at[idx_v], rows_v, sem).wait()
        pltpu.sync_copy(rows_v, out_hbm.at[pl.ds(base, bpw)])

    return k(table, idx_flat)


@jax.jit
def kernel(x, anchors):
    qsum = pl.pallas_call(
        _qsum_body,
        grid=(8,),
        in_specs=[pl.BlockSpec((8, T, D), lambda i: (i, 0, 0))],
        out_specs=pl.BlockSpec((8, D), lambda i: (i, 0)),
        out_shape=jax.ShapeDtypeStruct((BQ, D), jnp.float32),
    )(x)

    idx_pad = pl.pallas_call(
        _topk_body,
        grid=(NB,),
        in_specs=[
            pl.BlockSpec((BQ, D), lambda i: (0, 0)),
            pl.BlockSpec((BLK, D), lambda i: (i, 0)),
        ],
        out_specs=pl.BlockSpec((BQ, 128), lambda i: (0, 0)),
        out_shape=jax.ShapeDtypeStruct((BQ, 128), jnp.int32),
        scratch_shapes=[
            pltpu.VMEM((BQ, TOPK), jnp.float32),
            pltpu.VMEM((BQ, TOPK), jnp.int32),
        ],
    )(qsum, anchors)

    idx_flat = idx_pad[:, :TOPK].reshape(BQ * TOPK)
    gathered = _sc_gather(anchors, idx_flat)
    return jnp.concatenate([gathered.reshape(BQ, TOPK, D), x], axis=1)


# speedup vs baseline: 1.2765x; 1.0452x over previous
import functools

import jax
import jax.numpy as jnp
from jax import lax
from jax.experimental import pallas as pl
from jax.experimental.pallas import tpu as pltpu
from jax.experimental.pallas import tpu_sc as plsc

K_ANCH = 100000
D = 768
BQ = 64
T = 512
TOPK = 16
BLK = 2048
NB = (K_ANCH + BLK - 1) // BLK
IMIN = jnp.iinfo(jnp.int32).min
NEG = -3.0e38


def _mono(f):
    b = lax.bitcast_convert_type(f, jnp.int32)
    return jnp.where(b < 0, b ^ 0x7FFFFFFF, b)


def _unmono(b):
    return lax.bitcast_convert_type(
        jnp.where(b < 0, b ^ 0x7FFFFFFF, b), jnp.float32
    )


def _qsum_copy_body(x_ref, qs_ref, out_ref):
    x = x_ref[...]
    qs_ref[...] = jnp.sum(x, axis=1)[None]
    out_ref[:, TOPK:, :] = x
    out_ref[:, :TOPK, :] = jnp.zeros((x.shape[0], TOPK, D), jnp.float32)


def _insert_body(out_ref, g_ref, o_ref):
    del out_ref
    o_ref[...] = g_ref[...].reshape(o_ref.shape)


def _topk_body(qs_ref, a_ref, o_ref, rv_ref, ri_ref):
    i = pl.program_id(0)

    @pl.when(i == 0)
    def _init():
        rv_ref[...] = jnp.full((BQ, TOPK), NEG, jnp.float32)
        ri_ref[...] = jnp.zeros((BQ, TOPK), jnp.int32)

    a = a_ref[...]
    ss = jnp.sum(a * a, axis=1)
    inv = 1.0 / jnp.maximum(jnp.sqrt(ss), 1e-12)
    ab = a.astype(jnp.bfloat16)
    qb = qs_ref[...].astype(jnp.bfloat16)
    raw = lax.dot_general(
        qb, ab, (((1,), (1,)), ((), ())), preferred_element_type=jnp.float32
    )
    sims = raw * inv[None, :]
    lane = lax.broadcasted_iota(jnp.int32, (BQ, BLK), 1)
    sims = jnp.where(i * BLK + lane < K_ANCH, sims, NEG)

    keys = (_mono(sims) & ~(BLK - 1)) | (BLK - 1 - lane)
    bks = []
    for _ in range(TOPK):
        m = jnp.max(keys, axis=1, keepdims=True)
        keys = jnp.where(keys == m, IMIN, keys)
        bks.append(m)
    bk = jnp.concatenate(bks, axis=1)
    bi = i * BLK + (BLK - 1 - (bk & (BLK - 1)))
    bv = _unmono(bk & ~(BLK - 1))

    cv = jnp.concatenate([rv_ref[...], bv], axis=1)
    ci = jnp.concatenate([ri_ref[...], bi], axis=1)
    pos = lax.broadcasted_iota(jnp.int32, (BQ, 2 * TOPK), 1)
    mk = (_mono(cv) & ~31) | (31 - pos)
    nvs, nis = [], []
    for _ in range(TOPK):
        m = jnp.max(mk, axis=1, keepdims=True)
        hit = mk == m
        mk = jnp.where(hit, IMIN, mk)
        nvs.append(jnp.sum(jnp.where(hit, cv, 0.0), axis=1, keepdims=True))
        nis.append(jnp.sum(jnp.where(hit, ci, 0), axis=1, keepdims=True))
    rv_ref[...] = jnp.concatenate(nvs, axis=1)
    ri_ref[...] = jnp.concatenate(nis, axis=1)

    @pl.when(i == NB - 1)
    def _emit():
        o_ref[...] = jnp.concatenate(
            [ri_ref[...], jnp.zeros((BQ, 128 - TOPK), jnp.int32)], axis=1
        )


def _sc_gather(table, idx_flat):
    n = idx_flat.shape[0]
    bpw = n // 32
    mesh = plsc.VectorSubcoreMesh(core_axis_name="c", subcore_axis_name="s")

    @functools.partial(
        pl.kernel,
        mesh=mesh,
        out_type=jax.ShapeDtypeStruct((n, D), jnp.float32),
        scratch_types=[
            pltpu.VMEM((bpw,), jnp.int32),
            pltpu.VMEM((bpw, D), jnp.float32),
            pltpu.SemaphoreType.DMA,
        ],
    )
    def k(table_hbm, idx_hbm, out_hbm, idx_v, rows_v, sem):
        wid = lax.axis_index("s") * 2 + lax.axis_index("c")
        base = wid * bpw
        pltpu.sync_copy(idx_hbm.at[pl.ds(base, bpw)], idx_v)
        pltpu.async_copy(table_hbm.at[idx_v], rows_v, sem).wait()
        pltpu.sync_copy(rows_v, out_hbm.at[pl.ds(base, bpw)])

    return k(table, idx_flat)


@jax.jit
def kernel(x, anchors):
    qsum, outbuf = pl.pallas_call(
        _qsum_copy_body,
        grid=(16,),
        in_specs=[pl.BlockSpec((4, T, D), lambda i: (i, 0, 0))],
        out_specs=[
            pl.BlockSpec((1, 4, D), lambda i: (i, 0, 0)),
            pl.BlockSpec((4, T + TOPK, D), lambda i: (i, 0, 0)),
        ],
        out_shape=[
            jax.ShapeDtypeStruct((16, 4, D), jnp.float32),
            jax.ShapeDtypeStruct((BQ, T + TOPK, D), jnp.float32),
        ],
    )(x)
    qsum = qsum.reshape(BQ, D)

    idx_pad = pl.pallas_call(
        _topk_body,
        grid=(NB,),
        in_specs=[
            pl.BlockSpec((BQ, D), lambda i: (0, 0)),
            pl.BlockSpec((BLK, D), lambda i: (i, 0)),
        ],
        out_specs=pl.BlockSpec((BQ, 128), lambda i: (0, 0)),
        out_shape=jax.ShapeDtypeStruct((BQ, 128), jnp.int32),
        scratch_shapes=[
            pltpu.VMEM((BQ, TOPK), jnp.float32),
            pltpu.VMEM((BQ, TOPK), jnp.int32),
        ],
    )(qsum, anchors)

    idx_flat = idx_pad[:, :TOPK].reshape(BQ * TOPK)
    gathered = _sc_gather(anchors, idx_flat)

    return pl.pallas_call(
        _insert_body,
        grid=(16,),
        in_specs=[
            pl.BlockSpec((4, TOPK, D), lambda i: (i, 0, 0)),
            pl.BlockSpec((4 * TOPK, D), lambda i: (i, 0)),
        ],
        out_specs=pl.BlockSpec((4, TOPK, D), lambda i: (i, 0, 0)),
        out_shape=jax.ShapeDtypeStruct((BQ, T + TOPK, D), jnp.float32),
        input_output_aliases={0: 0},
    )(outbuf, gathered)


# speedup vs baseline: 1.3389x; 1.0488x over previous
import functools

import jax
import jax.numpy as jnp
from jax import lax
from jax.experimental import pallas as pl
from jax.experimental.pallas import tpu as pltpu
from jax.experimental.pallas import tpu_sc as plsc

K_ANCH = 100000
D = 768
BQ = 64
T = 512
TOPK = 16
BLK = 2048
NB = (K_ANCH + BLK - 1) // BLK
IMIN = jnp.iinfo(jnp.int32).min
NEG = -3.0e38


def _mono(f):
    b = lax.bitcast_convert_type(f, jnp.int32)
    return jnp.where(b < 0, b ^ 0x7FFFFFFF, b)


def _unmono(b):
    return lax.bitcast_convert_type(
        jnp.where(b < 0, b ^ 0x7FFFFFFF, b), jnp.float32
    )


def _qsum_body(x_ref, qs_ref):
    qs_ref[...] = jnp.sum(x_ref[...], axis=1)[None]


def _insert_body(out_ref, g_ref, o_ref):
    del out_ref
    o_ref[...] = g_ref[...].reshape(o_ref.shape)


def _topk_copy_body(qs_ref, a_ref, x_ref, o_ref, out_ref, rv_ref, ri_ref):
    i = pl.program_id(0)
    out_ref[...] = x_ref[...]

    @pl.when(i == 0)
    def _init():
        rv_ref[...] = jnp.full((BQ, TOPK), NEG, jnp.float32)
        ri_ref[...] = jnp.zeros((BQ, TOPK), jnp.int32)

    a = a_ref[...]
    ss = jnp.sum(a * a, axis=1)
    inv = 1.0 / jnp.maximum(jnp.sqrt(ss), 1e-12)
    ab = a.astype(jnp.bfloat16)
    qb = qs_ref[...].astype(jnp.bfloat16)
    raw = lax.dot_general(
        qb, ab, (((1,), (1,)), ((), ())), preferred_element_type=jnp.float32
    )
    sims = raw * inv[None, :]
    lane = lax.broadcasted_iota(jnp.int32, (BQ, BLK), 1)
    sims = jnp.where(i * BLK + lane < K_ANCH, sims, NEG)

    keys = (_mono(sims) & ~(BLK - 1)) | (BLK - 1 - lane)
    bks = []
    for _ in range(TOPK):
        m = jnp.max(keys, axis=1, keepdims=True)
        keys = jnp.where(keys == m, IMIN, keys)
        bks.append(m)
    bk = jnp.concatenate(bks, axis=1)
    bi = i * BLK + (BLK - 1 - (bk & (BLK - 1)))
    bv = _unmono(bk & ~(BLK - 1))

    cv = jnp.concatenate([rv_ref[...], bv], axis=1)
    ci = jnp.concatenate([ri_ref[...], bi], axis=1)
    pos = lax.broadcasted_iota(jnp.int32, (BQ, 2 * TOPK), 1)
    mk = (_mono(cv) & ~31) | (31 - pos)
    nvs, nis = [], []
    for _ in range(TOPK):
        m = jnp.max(mk, axis=1, keepdims=True)
        hit = mk == m
        mk = jnp.where(hit, IMIN, mk)
        nvs.append(jnp.sum(jnp.where(hit, cv, 0.0), axis=1, keepdims=True))
        nis.append(jnp.sum(jnp.where(hit, ci, 0), axis=1, keepdims=True))
    rv_ref[...] = jnp.concatenate(nvs, axis=1)
    ri_ref[...] = jnp.concatenate(nis, axis=1)

    @pl.when(i == NB - 1)
    def _emit():
        o_ref[...] = jnp.concatenate(
            [ri_ref[...], jnp.zeros((BQ, 128 - TOPK), jnp.int32)], axis=1
        )


def _sc_gather(table, idx_flat):
    n = idx_flat.shape[0]
    bpw = n // 32
    mesh = plsc.VectorSubcoreMesh(core_axis_name="c", subcore_axis_name="s")

    @functools.partial(
        pl.kernel,
        mesh=mesh,
        out_type=jax.ShapeDtypeStruct((n, D), jnp.float32),
        scratch_types=[
            pltpu.VMEM((bpw,), jnp.int32),
            pltpu.VMEM((bpw, D), jnp.float32),
            pltpu.SemaphoreType.DMA,
        ],
    )
    def k(table_hbm, idx_hbm, out_hbm, idx_v, rows_v, sem):
        wid = lax.axis_index("s") * 2 + lax.axis_index("c")
        base = wid * bpw
        pltpu.sync_copy(idx_hbm.at[pl.ds(base, bpw)], idx_v)
        pltpu.async_copy(table_hbm.at[idx_v], rows_v, sem).wait()
        pltpu.sync_copy(rows_v, out_hbm.at[pl.ds(base, bpw)])

    return k(table, idx_flat)


@jax.jit
def kernel(x, anchors):
    qsum = pl.pallas_call(
        _qsum_body,
        grid=(16,),
        in_specs=[pl.BlockSpec((4, T, D), lambda i: (i, 0, 0))],
        out_specs=pl.BlockSpec((1, 4, D), lambda i: (i, 0, 0)),
        out_shape=jax.ShapeDtypeStruct((16, 4, D), jnp.float32),
    )(x)
    qsum = qsum.reshape(BQ, D)

    XB = T // 32
    idx_pad, outbuf = pl.pallas_call(
        _topk_copy_body,
        grid=(NB,),
        in_specs=[
            pl.BlockSpec((BQ, D), lambda i: (0, 0)),
            pl.BlockSpec((BLK, D), lambda i: (i, 0)),
            pl.BlockSpec((BQ, XB, D), lambda i: (0, jnp.minimum(i, 31), 0)),
        ],
        out_specs=[
            pl.BlockSpec((BQ, 128), lambda i: (0, 0)),
            pl.BlockSpec((BQ, XB, D), lambda i: (0, jnp.minimum(i, 31) + 1, 0)),
        ],
        out_shape=[
            jax.ShapeDtypeStruct((BQ, 128), jnp.int32),
            jax.ShapeDtypeStruct((BQ, T + TOPK, D), jnp.float32),
        ],
        scratch_shapes=[
            pltpu.VMEM((BQ, TOPK), jnp.float32),
            pltpu.VMEM((BQ, TOPK), jnp.int32),
        ],
    )(qsum, anchors, x)

    idx_flat = idx_pad[:, :TOPK].reshape(BQ * TOPK)
    gathered = _sc_gather(anchors, idx_flat)

    return pl.pallas_call(
        _insert_body,
        grid=(16,),
        in_specs=[
            pl.BlockSpec((4, TOPK, D), lambda i: (i, 0, 0)),
            pl.BlockSpec((4 * TOPK, D), lambda i: (i, 0)),
        ],
        out_specs=pl.BlockSpec((4, TOPK, D), lambda i: (i, 0, 0)),
        out_shape=jax.ShapeDtypeStruct((BQ, T + TOPK, D), jnp.float32),
        input_output_aliases={0: 0},
    )(outbuf, gathered)


# speedup vs baseline: 1.8496x; 1.3814x over previous
import functools

import jax
import jax.numpy as jnp
from jax import lax
from jax.experimental import pallas as pl
from jax.experimental.pallas import tpu as pltpu
from jax.experimental.pallas import tpu_sc as plsc

K_ANCH = 100000
D = 768
BQ = 64
T = 512
TOPK = 16
BLK = 2048
NB = (K_ANCH + BLK - 1) // BLK
NX = 32
XB = T // NX
IMIN = jnp.iinfo(jnp.int32).min
NEG = -3.0e38
LOW = 4096 - 1


def _mono(f):
    b = lax.bitcast_convert_type(f, jnp.int32)
    return jnp.where(b < 0, b ^ 0x7FFFFFFF, b)


def _unmono(b):
    return lax.bitcast_convert_type(
        jnp.where(b < 0, b ^ 0x7FFFFFFF, b), jnp.float32
    )


def _insert_body(out_ref, g_ref, o_ref):
    del out_ref
    o_ref[...] = g_ref[...].reshape(o_ref.shape)


def _main_body(a_ref, x_ref, o_ref, out_ref, qs_ref, rv_ref, ri_ref):
    i = pl.program_id(0)

    @pl.when(i < NX)
    def _phase1():
        x = x_ref[...]
        out_ref[...] = x
        s = jnp.sum(x, axis=1)
        qs_ref[...] = jnp.where(i == 0, s, qs_ref[...] + s)

    @pl.when(i >= NX)
    def _phase2():
        j = i - NX

        @pl.when(j == 0)
        def _init():
            rv_ref[...] = jnp.full((BQ, TOPK), NEG, jnp.float32)
            ri_ref[...] = jnp.zeros((BQ, TOPK), jnp.int32)

        a = a_ref[...]
        ss = jnp.sum(a * a, axis=1)
        inv = lax.rsqrt(jnp.maximum(ss, 1e-24))
        ab = a.astype(jnp.bfloat16)
        qb = qs_ref[...].astype(jnp.bfloat16)
        raw = lax.dot_general(
            qb, ab, (((1,), (1,)), ((), ())),
            preferred_element_type=jnp.float32,
        )
        sims = raw * inv[None, :]
        col = lax.broadcasted_iota(jnp.int32, (BQ, BLK), 1)
        sims = jnp.where(j * BLK + col < K_ANCH, sims, NEG)

        bkeys = (_mono(sims) & ~LOW) | (2047 - col)
        pos = lax.broadcasted_iota(jnp.int32, (BQ, TOPK), 1)
        rkeys = (_mono(rv_ref[...]) & ~LOW) | (2175 - pos)
        pad = jnp.concatenate(
            [rkeys, jnp.full((BQ, 128 - TOPK), IMIN, jnp.int32)], axis=1
        )
        keys = jnp.concatenate([pad, bkeys], axis=1)

        bks = []
        for _ in range(TOPK):
            m = jnp.max(keys, axis=1, keepdims=True)
            keys = jnp.where(keys == m, IMIN, keys)
            bks.append(m)
        bk = jnp.concatenate(bks, axis=1)

        lanes = 2175 - (bk & LOW)
        is_blk = lanes >= 128
        idx_blk = j * BLK + (lanes - 128)
        ri = ri_ref[...]
        idx_run = jnp.zeros((BQ, TOPK), jnp.int32)
        for t in range(TOPK):
            idx_run = jnp.where(lanes == t, ri[:, t : t + 1], idx_run)
        ri_ref[...] = jnp.where(is_blk, idx_blk, idx_run)
        rv_ref[...] = _unmono(bk & ~LOW)

        @pl.when(j == NB - 1)
        def _emit():
            o_ref[...] = jnp.concatenate(
                [ri_ref[...], jnp.zeros((BQ, 128 - TOPK), jnp.int32)],
                axis=1,
            )


def _sc_gather(table, idx_flat):
    n = idx_flat.shape[0]
    bpw = n // 32
    mesh = plsc.VectorSubcoreMesh(core_axis_name="c", subcore_axis_name="s")

    @functools.partial(
        pl.kernel,
        mesh=mesh,
        out_type=jax.ShapeDtypeStruct((n, D), jnp.float32),
        scratch_types=[
            pltpu.VMEM((bpw,), jnp.int32),
            pltpu.VMEM((bpw, D), jnp.float32),
            pltpu.SemaphoreType.DMA,
        ],
    )
    def k(table_hbm, idx_hbm, out_hbm, idx_v, rows_v, sem):
        wid = lax.axis_index("s") * 2 + lax.axis_index("c")
        base = wid * bpw
        pltpu.sync_copy(idx_hbm.at[pl.ds(base, bpw)], idx_v)
        pltpu.async_copy(table_hbm.at[idx_v], rows_v, sem).wait()
        pltpu.sync_copy(rows_v, out_hbm.at[pl.ds(base, bpw)])

    return k(table, idx_flat)


@jax.jit
def kernel(x, anchors):
    idx_pad, outbuf = pl.pallas_call(
        _main_body,
        grid=(NX + NB,),
        in_specs=[
            pl.BlockSpec((BLK, D), lambda i: (jnp.maximum(i - NX, 0), 0)),
            pl.BlockSpec(
                (BQ, XB, D), lambda i: (0, jnp.minimum(i, NX - 1), 0)
            ),
        ],
        out_specs=[
            pl.BlockSpec((BQ, 128), lambda i: (0, 0)),
            pl.BlockSpec(
                (BQ, XB, D), lambda i: (0, jnp.minimum(i, NX - 1) + 1, 0)
            ),
        ],
        out_shape=[
            jax.ShapeDtypeStruct((BQ, 128), jnp.int32),
            jax.ShapeDtypeStruct((BQ, T + TOPK, D), jnp.float32),
        ],
        scratch_shapes=[
            pltpu.VMEM((BQ, D), jnp.float32),
            pltpu.VMEM((BQ, TOPK), jnp.float32),
            pltpu.VMEM((BQ, TOPK), jnp.int32),
        ],
    )(anchors, x)

    idx_flat = idx_pad[:, :TOPK].reshape(BQ * TOPK)
    gathered = _sc_gather(anchors, idx_flat)

    return pl.pallas_call(
        _insert_body,
        grid=(16,),
        in_specs=[
            pl.BlockSpec((4, TOPK, D), lambda i: (i, 0, 0)),
            pl.BlockSpec((4 * TOPK, D), lambda i: (i, 0)),
        ],
        out_specs=pl.BlockSpec((4, TOPK, D), lambda i: (i, 0, 0)),
        out_shape=jax.ShapeDtypeStruct((BQ, T + TOPK, D), jnp.float32),
        input_output_aliases={0: 0},
    )(outbuf, gathered)
